# Initial kernel scaffold; baseline (speedup 1.0000x reference)
import jax
import jax.numpy as jnp
from jax.experimental import pallas as pl


def kernel(x, edge_index, batch, Wl, bl, Wr, Wlin, blin):
    raise NotImplementedError("write your pallas kernel here")


# speedup vs baseline: 5.8293x; 5.8293x over previous
import functools

import jax
import jax.numpy as jnp
from jax import lax
from jax.experimental import pallas as pl
from jax.experimental.pallas import tpu as pltpu
from jax.experimental.pallas import tpu_sc as plsc

N = 10000
NP = 10240
E = 320000
D = 128
G = 64
NC = 2
NS = 16
NW = NC * NS
EPT = E // NW
CH = 80
NCH = EPT // CH
RPT = NP // NS

HI = NP // D
EB = 2000
NEB = E // EB

BLK = 2000
NB = N // BLK


def _sc_agg_body(x_hbm, src_hbm, dst_hbm, out_sum,
                 acc, src_idx, dst_idx, rows,
                 sem_z, sem_i1, sem_i2, sem_g, sem_a):
    cid = lax.axis_index("c")
    sid = lax.axis_index("s")
    w = cid * NS + sid

    z16 = jnp.zeros((16,), jnp.float32)

    @pl.loop(0, CH)
    def _(i):
        for j in range(D // 16):
            rows[i, pl.ds(j * 16, 16)] = z16

    @pl.loop(0, RPT // CH)
    def _(k):
        z = pl.multiple_of(sid * RPT + k * CH, 8)
        pltpu.async_copy(rows, acc.at[pl.ds(z, CH)], sem_z).wait()

    plsc.subcore_barrier()

    e0 = w * EPT

    @pl.loop(0, NCH)
    def _(i):
        base = pl.multiple_of(e0 + i * CH, 8)
        pltpu.async_copy(src_hbm.at[pl.ds(base, CH)], src_idx, sem_i1).wait()
        pltpu.async_copy(dst_hbm.at[pl.ds(base, CH)], dst_idx, sem_i2).wait()
        pltpu.async_copy(x_hbm.at[src_idx], rows, sem_g).wait()
        pltpu.async_copy(rows, acc.at[dst_idx], sem_a, add=True).wait()

    plsc.subcore_barrier()

    r0 = pl.multiple_of(sid * RPT, 8)
    pltpu.async_copy(acc.at[pl.ds(r0, RPT)],
                     out_sum.at[cid, pl.ds(r0, RPT)], sem_z).wait()


_sc_agg = functools.partial(
    pl.kernel,
    out_type=jax.ShapeDtypeStruct((NC, NP, D), jnp.float32),
    mesh=plsc.VectorSubcoreMesh(core_axis_name="c", subcore_axis_name="s"),
    scratch_types=[
        pltpu.VMEM_SHARED((NP, D), jnp.float32),
        pltpu.VMEM((CH,), jnp.int32),
        pltpu.VMEM((CH,), jnp.int32),
        pltpu.VMEM((CH, D), jnp.float32),
    ] + [pltpu.SemaphoreType.DMA] * 5,
)(_sc_agg_body)


def _cnt_body(dst_ref, cnt_ref):
    i = pl.program_id(0)

    @pl.when(i == 0)
    def _():
        cnt_ref[...] = jnp.zeros_like(cnt_ref)

    d = dst_ref[...]
    hi = d // D
    lo = d - hi * D
    ahi = (hi == lax.broadcasted_iota(jnp.int32, (EB, HI), 1)
           ).astype(jnp.bfloat16)
    alo = (lo == lax.broadcasted_iota(jnp.int32, (EB, D), 1)
           ).astype(jnp.bfloat16)
    cnt_ref[...] += lax.dot_general(
        ahi, alo, (((0,), (0,)), ((), ())),
        preferred_element_type=jnp.float32)


def _tc_cnt(dst2):
    return pl.pallas_call(
        _cnt_body,
        grid=(NEB,),
        in_specs=[pl.BlockSpec((EB, 1), lambda i: (i, 0))],
        out_specs=pl.BlockSpec((HI, D), lambda i: (0, 0)),
        out_shape=jax.ShapeDtypeStruct((HI, D), jnp.float32),
    )(dst2)


def _tc_body(sum_ref, cnt_ref, x_ref, bvec_ref, bsmem_ref,
             wlT_ref, wrT_ref, bl_ref, wlinT_ref, blin_ref,
             out_ref, acc_ref):
    i = pl.program_id(0)

    @pl.when(i == 0)
    def _():
        acc_ref[...] = jnp.zeros_like(acc_ref)

    summed = sum_ref[0] + sum_ref[1]
    cnt = cnt_ref[...]
    mean = summed / jnp.maximum(cnt, 1.0)
    h = (jnp.dot(mean, wlT_ref[...], preferred_element_type=jnp.float32)
         + jnp.dot(x_ref[...], wrT_ref[...], preferred_element_type=jnp.float32)
         + bl_ref[...])
    h = jnp.maximum(h, 0.0)

    bvec = bvec_ref[...]
    g0 = bsmem_ref[i * BLK]
    g1 = bsmem_ref[i * BLK + BLK - 1]
    rows_iota = lax.broadcasted_iota(jnp.int32, (G, 1), 0)

    def pool_body(g, c):
        m = bvec == g
        contrib = jnp.max(jnp.where(m, h, 0.0), axis=0, keepdims=True)
        upd = jnp.where(rows_iota == g, contrib, 0.0)
        acc_ref[...] = jnp.maximum(acc_ref[...], upd)
        return c
    lax.fori_loop(g0, g1 + 1, pool_body, 0)

    @pl.when(i == NB - 1)
    def _():
        out_ref[...] = (
            jnp.dot(acc_ref[...], wlinT_ref[...],
                    preferred_element_type=jnp.float32)
            + blin_ref[...])


def _tc_finish(parts_sum, cnt, x, bvec, batch, wlT, wrT, bl2,
               wlinT, blin2):
    return pl.pallas_call(
        _tc_body,
        grid=(NB,),
        in_specs=[
            pl.BlockSpec((NC, BLK, D), lambda i: (0, i, 0)),
            pl.BlockSpec((BLK, 1), lambda i: (i, 0)),
            pl.BlockSpec((BLK, D), lambda i: (i, 0)),
            pl.BlockSpec((BLK, 1), lambda i: (i, 0)),
            pl.BlockSpec(memory_space=pltpu.SMEM),
            pl.BlockSpec((D, D), lambda i: (0, 0)),
            pl.BlockSpec((D, D), lambda i: (0, 0)),
            pl.BlockSpec((1, D), lambda i: (0, 0)),
            pl.BlockSpec((D, 2), lambda i: (0, 0)),
            pl.BlockSpec((1, 2), lambda i: (0, 0)),
        ],
        out_specs=pl.BlockSpec((G, 2), lambda i: (0, 0)),
        out_shape=jax.ShapeDtypeStruct((G, 2), jnp.float32),
        scratch_shapes=[pltpu.VMEM((G, D), jnp.float32)],
    )(parts_sum, cnt, x, bvec, batch, wlT, wrT, bl2, wlinT, blin2)


def kernel(x, edge_index, batch, Wl, bl, Wr, Wlin, blin):
    src = edge_index[0]
    dst = edge_index[1]
    parts_sum = _sc_agg(x, src, dst)
    cnt2d = _tc_cnt(dst.reshape(E, 1))
    cnt = cnt2d.reshape(NP)[:N].reshape(N, 1)
    return _tc_finish(parts_sum, cnt, x, batch.reshape(N, 1), batch,
                      Wl.T, Wr.T, bl.reshape(1, D), Wlin.T,
                      blin.reshape(1, 2))


# speedup vs baseline: 6.4331x; 1.1036x over previous
import functools

import jax
import jax.numpy as jnp
from jax import lax
from jax.experimental import pallas as pl
from jax.experimental.pallas import tpu as pltpu
from jax.experimental.pallas import tpu_sc as plsc

N = 10000
NP = 10240
E = 320000
D = 128
G = 64
NC = 2
NS = 16
NW = NC * NS
EPT = E // NW
CH = 80
NCH = EPT // CH
RPT = NP // NS

HI = NP // D
EB = 2000
NEB = E // EB

BLK = 2000
NB = N // BLK


def _sc_agg_body(x_hbm, src_hbm, dst_hbm, out_sum,
                 acc, src_a, dst_a, src_b, dst_b, rows_a, rows_b,
                 sem_i, sem_ga, sem_gb):
    cid = lax.axis_index("c")
    sid = lax.axis_index("s")
    w = cid * NS + sid

    z16 = jnp.zeros((16,), jnp.float32)

    @pl.loop(0, CH)
    def _(i):
        for j in range(D // 16):
            rows_a[i, pl.ds(j * 16, 16)] = z16

    @pl.loop(0, RPT // CH)
    def _(k):
        z = pl.multiple_of(sid * RPT + k * CH, 8)
        pltpu.async_copy(rows_a, acc.at[pl.ds(z, CH)], sem_i).wait()

    plsc.subcore_barrier()

    e0 = w * EPT

    def _load_idx(c, sbuf, dbuf):
        base = pl.multiple_of(e0 + c * CH, 8)
        cp1 = pltpu.async_copy(src_hbm.at[pl.ds(base, CH)], sbuf, sem_i)
        cp2 = pltpu.async_copy(dst_hbm.at[pl.ds(base, CH)], dbuf, sem_i)
        cp1.wait()
        cp2.wait()

    _load_idx(0, src_a, dst_a)
    ga0 = pltpu.async_copy(x_hbm.at[src_a], rows_a, sem_ga)

    @pl.loop(0, (NCH - 1) // 2)
    def _(i):
        c1 = 2 * i + 1
        _load_idx(c1, src_b, dst_b)
        gb = pltpu.async_copy(x_hbm.at[src_b], rows_b, sem_gb)
        pltpu.make_async_copy(x_hbm.at[src_a], rows_a, sem_ga).wait()
        pltpu.async_copy(rows_a, acc.at[dst_a], sem_i, add=True).wait()

        c2 = 2 * i + 2
        _load_idx(c2, src_a, dst_a)
        pltpu.async_copy(x_hbm.at[src_a], rows_a, sem_ga)
        gb.wait()
        pltpu.async_copy(rows_b, acc.at[dst_b], sem_i, add=True).wait()

    pltpu.make_async_copy(x_hbm.at[src_a], rows_a, sem_ga).wait()
    pltpu.async_copy(rows_a, acc.at[dst_a], sem_i, add=True).wait()

    plsc.subcore_barrier()

    r0 = pl.multiple_of(sid * RPT, 8)
    pltpu.async_copy(acc.at[pl.ds(r0, RPT)],
                     out_sum.at[cid, pl.ds(r0, RPT)], sem_i).wait()


_sc_agg = functools.partial(
    pl.kernel,
    out_type=jax.ShapeDtypeStruct((NC, NP, D), jnp.float32),
    mesh=plsc.VectorSubcoreMesh(core_axis_name="c", subcore_axis_name="s"),
    scratch_types=[
        pltpu.VMEM_SHARED((NP, D), jnp.float32),
        pltpu.VMEM((CH,), jnp.int32),
        pltpu.VMEM((CH,), jnp.int32),
        pltpu.VMEM((CH,), jnp.int32),
        pltpu.VMEM((CH,), jnp.int32),
        pltpu.VMEM((CH, D), jnp.float32),
        pltpu.VMEM((CH, D), jnp.float32),
    ] + [pltpu.SemaphoreType.DMA] * 3,
)(_sc_agg_body)


def _cnt_body(dst_ref, cnt_ref):
    i = pl.program_id(0)

    @pl.when(i == 0)
    def _():
        cnt_ref[...] = jnp.zeros_like(cnt_ref)

    d = dst_ref[...]
    hi = d // D
    lo = d - hi * D
    ahi = (hi == lax.broadcasted_iota(jnp.int32, (EB, HI), 1)
           ).astype(jnp.bfloat16)
    alo = (lo == lax.broadcasted_iota(jnp.int32, (EB, D), 1)
           ).astype(jnp.bfloat16)
    cnt_ref[...] += lax.dot_general(
        ahi, alo, (((0,), (0,)), ((), ())),
        preferred_element_type=jnp.float32)


def _tc_cnt(dst2):
    return pl.pallas_call(
        _cnt_body,
        grid=(NEB,),
        in_specs=[pl.BlockSpec((EB, 1), lambda i: (i, 0))],
        out_specs=pl.BlockSpec((HI, D), lambda i: (0, 0)),
        out_shape=jax.ShapeDtypeStruct((HI, D), jnp.float32),
    )(dst2)


def _tc_body(sum_ref, cnt_ref, x_ref, bvec_ref, bsmem_ref,
             wlT_ref, wrT_ref, bl_ref, wlinT_ref, blin_ref,
             out_ref, acc_ref):
    i = pl.program_id(0)

    @pl.when(i == 0)
    def _():
        acc_ref[...] = jnp.zeros_like(acc_ref)

    summed = sum_ref[0] + sum_ref[1]
    cnt = cnt_ref[...]
    mean = summed / jnp.maximum(cnt, 1.0)
    h = (jnp.dot(mean, wlT_ref[...], preferred_element_type=jnp.float32)
         + jnp.dot(x_ref[...], wrT_ref[...], preferred_element_type=jnp.float32)
         + bl_ref[...])
    h = jnp.maximum(h, 0.0)

    bvec = bvec_ref[...]
    g0 = bsmem_ref[i * BLK]
    g1 = bsmem_ref[i * BLK + BLK - 1]
    rows_iota = lax.broadcasted_iota(jnp.int32, (G, 1), 0)

    def pool_body(g, c):
        m = bvec == g
        contrib = jnp.max(jnp.where(m, h, 0.0), axis=0, keepdims=True)
        upd = jnp.where(rows_iota == g, contrib, 0.0)
        acc_ref[...] = jnp.maximum(acc_ref[...], upd)
        return c
    lax.fori_loop(g0, g1 + 1, pool_body, 0)

    @pl.when(i == NB - 1)
    def _():
        out_ref[...] = (
            jnp.dot(acc_ref[...], wlinT_ref[...],
                    preferred_element_type=jnp.float32)
            + blin_ref[...])


def _tc_finish(parts_sum, cnt, x, bvec, batch, wlT, wrT, bl2,
               wlinT, blin2):
    return pl.pallas_call(
        _tc_body,
        grid=(NB,),
        in_specs=[
            pl.BlockSpec((NC, BLK, D), lambda i: (0, i, 0)),
            pl.BlockSpec((BLK, 1), lambda i: (i, 0)),
            pl.BlockSpec((BLK, D), lambda i: (i, 0)),
            pl.BlockSpec((BLK, 1), lambda i: (i, 0)),
            pl.BlockSpec(memory_space=pltpu.SMEM),
            pl.BlockSpec((D, D), lambda i: (0, 0)),
            pl.BlockSpec((D, D), lambda i: (0, 0)),
            pl.BlockSpec((1, D), lambda i: (0, 0)),
            pl.BlockSpec((D, 2), lambda i: (0, 0)),
            pl.BlockSpec((1, 2), lambda i: (0, 0)),
        ],
        out_specs=pl.BlockSpec((G, 2), lambda i: (0, 0)),
        out_shape=jax.ShapeDtypeStruct((G, 2), jnp.float32),
        scratch_shapes=[pltpu.VMEM((G, D), jnp.float32)],
    )(parts_sum, cnt, x, bvec, batch, wlT, wrT, bl2, wlinT, blin2)


def kernel(x, edge_index, batch, Wl, bl, Wr, Wlin, blin):
    src = edge_index[0]
    dst = edge_index[1]
    parts_sum = _sc_agg(x, src, dst)
    cnt2d = _tc_cnt(dst.reshape(E, 1))
    cnt = cnt2d.reshape(NP)[:N].reshape(N, 1)
    return _tc_finish(parts_sum, cnt, x, batch.reshape(N, 1), batch,
                      Wl.T, Wr.T, bl.reshape(1, D), Wlin.T,
                      blin.reshape(1, 2))
